# baseline (speedup 1.0000x reference)
import functools

import jax
import jax.numpy as jnp
import numpy as np
from jax import lax
from jax.experimental import pallas as pl
from jax.experimental.pallas import tpu as pltpu
from jax.experimental.pallas import tpu_sc as plsc

N = 2560
E = 163840
NH = 4
HD = 64
D = NH * HD

NC = 2
NS = 16
NW = NC * NS
ROWS_PER_W = N // NW
SC_PASSES = 2
ROWS_PER_PASS = ROWS_PER_W // SC_PASSES
CHUNK = 4096
NCHUNK = E // CHUNK


UNROLL = 16


def _count_body(src_hbm, dst_hbm, c_hbm, src_v, dst_v, cnt_v, sems, semd):
    wid = lax.axis_index("s") * NC + lax.axis_index("c")
    ones = jnp.ones((16,), jnp.float32)
    zeros16 = jnp.zeros((16,), jnp.float32)

    def start(ci, b):
        pltpu.async_copy(src_hbm.at[pl.ds(ci * CHUNK, CHUNK)],
                         src_v.at[b], sems.at[b])
        pltpu.async_copy(dst_hbm.at[pl.ds(ci * CHUNK, CHUNK)],
                         dst_v.at[b], semd.at[b])

    def wait(b):
        pltpu.make_async_copy(src_hbm.at[pl.ds(0, CHUNK)],
                              src_v.at[b], sems.at[b]).wait()
        pltpu.make_async_copy(dst_hbm.at[pl.ds(0, CHUNK)],
                              dst_v.at[b], semd.at[b]).wait()

    for p in range(SC_PASSES):
        row_base = wid * ROWS_PER_W + p * ROWS_PER_PASS

        @plsc.parallel_loop(0, ROWS_PER_PASS * N // 16, step=1, unroll=UNROLL)
        def _(i):
            cnt_v[pl.ds(i * 16, 16)] = zeros16

        start(0, 0)
        start(1, 1)

        def chunk_pair(ci, carry):
            for b in range(2):
                wait(b)

                @plsc.parallel_loop(0, CHUNK // 16, step=1, unroll=UNROLL)
                def _(i):
                    s = src_v[b, pl.ds(i * 16, 16)]
                    d = dst_v[b, pl.ds(i * 16, 16)]
                    r = d - row_base
                    m = (r >= 0) & (r < ROWS_PER_PASS)
                    fl = jnp.where(m, r * N + s, 0)
                    plsc.addupdate_scatter(cnt_v, [fl], ones, mask=m)

                @pl.when(ci + 2 + b < NCHUNK)
                def _():
                    start(ci + 2 + b, b)
            return carry

        lax.fori_loop(0, NCHUNK // 2, lambda i, c: chunk_pair(i * 2, c), 0)
        pltpu.sync_copy(
            cnt_v, c_hbm.at[pl.ds(row_base * N, ROWS_PER_PASS * N)])


def _build_counts(src, dst):
    mesh = plsc.VectorSubcoreMesh(
        core_axis_name="c", subcore_axis_name="s",
        num_cores=NC, num_subcores=NS)
    kern = pl.kernel(
        _count_body,
        out_type=jax.ShapeDtypeStruct((N * N,), jnp.float32),
        mesh=mesh,
        scratch_types=[
            pltpu.VMEM((2, CHUNK), jnp.int32),
            pltpu.VMEM((2, CHUNK), jnp.int32),
            pltpu.VMEM((ROWS_PER_PASS * N,), jnp.float32),
            pltpu.SemaphoreType.DMA((2,)),
            pltpu.SemaphoreType.DMA((2,)),
        ],
        compiler_params=pltpu.CompilerParams(needs_layout_passes=False),
    )
    return kern(src, dst).reshape(N, N)


def _prep1_body(x_ref, w_ref, amat_ref, h_ref, sc_ref):
    h = jnp.dot(x_ref[...], w_ref[...], preferred_element_type=jnp.float32)
    h_ref[...] = h
    sc_ref[...] = jnp.dot(h, amat_ref[...], preferred_element_type=jnp.float32,
                          precision=lax.Precision.HIGHEST)


def _prep_bn_body(raw_ref, g_ref, be_ref, w_ref, amat_ref, h_ref, sc_ref):
    r = raw_ref[...]
    m = jnp.mean(r, axis=0, keepdims=True)
    v = jnp.mean((r - m) * (r - m), axis=0, keepdims=True)
    xn = (r - m) * lax.rsqrt(v + 1e-5) * g_ref[...] + be_ref[...]
    xn = jnp.maximum(xn, 0.0)
    h = jnp.dot(xn, w_ref[...], preferred_element_type=jnp.float32)
    h_ref[...] = h
    sc_ref[...] = jnp.dot(h, amat_ref[...], preferred_element_type=jnp.float32,
                          precision=lax.Precision.HIGHEST)


def _prep(x, w, amat, g=None, be=None):
    n, f = x.shape
    body = _prep1_body if g is None else _prep_bn_body
    args = (x, w, amat) if g is None else (x, g.reshape(1, D), be.reshape(1, D), w, amat)
    return pl.pallas_call(
        body,
        out_shape=(jax.ShapeDtypeStruct((n, D), jnp.float32),
                   jax.ShapeDtypeStruct((n, 2 * NH), jnp.float32)),
    )(*args)


TD = 640


def _att_body(c_ref, sct_ref, sc_ref, h_ref, b_ref, out_ref):
    i = pl.program_id(0)
    cmat = c_ref[...]
    cols = []
    for hd in range(NH):
        arow = sct_ref[hd:hd + 1, :]
        acol = sc_ref[pl.ds(i * TD, TD), NH + hd:NH + hd + 1]
        mxs = jnp.max(arow, axis=1, keepdims=True)
        al = arow + acol
        al = jnp.where(al >= 0.0, al, 0.2 * al)
        amax = mxs + acol
        amax = jnp.where(amax >= 0.0, amax, 0.2 * amax)
        ex = cmat * jnp.exp(al - amax)
        denom = jnp.sum(ex, axis=1, keepdims=True)
        hf = h_ref[:, hd * HD:(hd + 1) * HD]
        hh = hf.astype(jnp.bfloat16)
        hl = (hf - hh.astype(jnp.float32)).astype(jnp.bfloat16)
        a16 = ex.astype(jnp.bfloat16)
        b16 = (ex - a16.astype(jnp.float32)).astype(jnp.bfloat16)
        num = (jnp.dot(a16, hh, preferred_element_type=jnp.float32)
               + jnp.dot(b16, hh, preferred_element_type=jnp.float32)
               + jnp.dot(a16, hl, preferred_element_type=jnp.float32))
        cols.append(num / (denom + 1e-16))
    out_ref[...] = jnp.concatenate(cols, axis=1) + b_ref[...]


def _attention(c, sct, sc, h, b):
    b = b.reshape(1, D)
    grid = (N // TD,)
    return pl.pallas_call(
        _att_body,
        grid=grid,
        in_specs=[
            pl.BlockSpec((TD, N), lambda i: (i, 0)),
            pl.BlockSpec((8, N), lambda i: (0, 0)),
            pl.BlockSpec((N, 2 * NH), lambda i: (0, 0)),
            pl.BlockSpec((N, D), lambda i: (0, 0)),
            pl.BlockSpec((1, D), lambda i: (0, 0)),
        ],
        out_specs=pl.BlockSpec((TD, D), lambda i: (i, 0)),
        out_shape=jax.ShapeDtypeStruct((N, D), jnp.float32),
    )(c, sct, sc, h, b)


def _final_body(raw_ref, g_ref, be_ref, wfc_ref, bfc_ref, mask_ref,
                tri_ref, adj_ref, sel_ref, selo_ref, out_ref):
    r = raw_ref[...]
    m = jnp.mean(r, axis=0, keepdims=True)
    v = jnp.mean((r - m) * (r - m), axis=0, keepdims=True)
    xn = (r - m) * lax.rsqrt(v + 1e-5) * g_ref[...] + be_ref[...]
    xn = jnp.maximum(xn, 0.0)
    t = jnp.dot(xn, wfc_ref[...], preferred_element_type=jnp.float32)
    t = (t + bfc_ref[...]) * mask_ref[...]
    tri = tri_ref[...]
    adj = adj_ref[...]
    a = [jnp.dot(t, sel_ref[k], preferred_element_type=jnp.float32) * tri
         for k in range(3)]
    mx = jnp.maximum(jnp.maximum(a[0], a[1]), a[2])
    e = [jnp.exp(ak - mx) for ak in a]
    s = e[0] + e[1] + e[2]
    p0 = jnp.where(adj == 0.0, 0.0, e[0] / s)
    p1 = e[1] / s
    p2 = jnp.where(adj == 1.0, 0.0, e[2] / s)
    tot = p0 + p1 + p2
    out = (jnp.dot(p0 / tot, selo_ref[0], preferred_element_type=jnp.float32)
           + jnp.dot(p1 / tot, selo_ref[1], preferred_element_type=jnp.float32)
           + jnp.dot(p2 / tot, selo_ref[2], preferred_element_type=jnp.float32))
    out_ref[...] = out


def _final(raw, g, be, wfc_p, bfc_p, mask2, tri_r, adj_r, sel, selo):
    return pl.pallas_call(
        _final_body,
        out_shape=jax.ShapeDtypeStruct((N, 128), jnp.float32),
    )(raw, g.reshape(1, D), be.reshape(1, D), wfc_p, bfc_p, mask2,
      tri_r, adj_r, sel, selo)


def _amat(a_src, a_dst):
    eye = jnp.eye(NH, dtype=jnp.float32)
    blk_s = jnp.einsum("hc,hk->hck", a_src.reshape(NH, HD), eye).reshape(D, NH)
    blk_d = jnp.einsum("hc,hk->hck", a_dst.reshape(NH, HD), eye).reshape(D, NH)
    return jnp.concatenate([blk_s, blk_d], axis=1)


def kernel(x, edge_index, pos, mask, adjacency, tri, W1, as1, ad1, b1, g1, be1,
           W2, as2, ad2, b2, g2, be2, W3, as3, ad3, b3, g3, be3, Wfc, bfc):
    src = edge_index[0]
    dst = edge_index[1]
    c = _build_counts(src, dst)

    sel_np = np.zeros((3, 128, 40), np.float32)
    for k in range(3):
        sel_np[k, np.arange(40) * 3 + k, np.arange(40)] = 1.0
    sel = jnp.asarray(sel_np)
    selo = jnp.asarray(np.transpose(sel_np, (0, 2, 1)))

    wfc_p = jnp.zeros((D, 128), jnp.float32).at[:, :120].set(Wfc)
    bfc_p = jnp.zeros((1, 128), jnp.float32).at[:, :120].set(bfc)
    mask2 = mask.reshape(N, 1)
    tri_r = tri.reshape(N, 40)
    adj_r = adjacency.astype(jnp.float32).reshape(N, 40)

    h, sc = _prep(x, W1, _amat(as1, ad1))
    sct = jnp.zeros((8, N), jnp.float32).at[:4, :].set(sc[:, :4].T)
    raw = _attention(c, sct, sc, h, b1)
    h, sc = _prep(raw, W2, _amat(as2, ad2), g1, be1)
    sct = jnp.zeros((8, N), jnp.float32).at[:4, :].set(sc[:, :4].T)
    raw = _attention(c, sct, sc, h, b2)
    h, sc = _prep(raw, W3, _amat(as3, ad3), g2, be2)
    sct = jnp.zeros((8, N), jnp.float32).at[:4, :].set(sc[:, :4].T)
    raw = _attention(c, sct, sc, h, b3)
    out = _final(raw, g3, be3, wfc_p, bfc_p, mask2, tri_r, adj_r, sel, selo)
    return out[:, :120].reshape(64, 40, 40, 3)


# speedup vs baseline: 1.0238x; 1.0238x over previous
import functools

import jax
import jax.numpy as jnp
import numpy as np
from jax import lax
from jax.experimental import pallas as pl
from jax.experimental.pallas import tpu as pltpu
from jax.experimental.pallas import tpu_sc as plsc

N = 2560
E = 163840
NH = 4
HD = 64
D = NH * HD

NC = 2
NS = 16
NW = NC * NS
ROWS_PER_W = N // NW
SC_PASSES = 2
ROWS_PER_PASS = ROWS_PER_W // SC_PASSES
CHUNK = 4096
NCHUNK = E // CHUNK


UNROLL = 8


def _count_body(src_hbm, dst_hbm, c_hbm, src_v, dst_v, cnt_v, sems, semd):
    wid = lax.axis_index("s") * NC + lax.axis_index("c")
    ones = jnp.ones((16,), jnp.float32)
    zeros16 = jnp.zeros((16,), jnp.float32)

    def start(ci, b):
        pltpu.async_copy(src_hbm.at[pl.ds(ci * CHUNK, CHUNK)],
                         src_v.at[b], sems.at[b])
        pltpu.async_copy(dst_hbm.at[pl.ds(ci * CHUNK, CHUNK)],
                         dst_v.at[b], semd.at[b])

    def wait(b):
        pltpu.make_async_copy(src_hbm.at[pl.ds(0, CHUNK)],
                              src_v.at[b], sems.at[b]).wait()
        pltpu.make_async_copy(dst_hbm.at[pl.ds(0, CHUNK)],
                              dst_v.at[b], semd.at[b]).wait()

    for p in range(SC_PASSES):
        row_base = wid * ROWS_PER_W + p * ROWS_PER_PASS

        @plsc.parallel_loop(0, ROWS_PER_PASS * N // 16, step=1, unroll=UNROLL)
        def _(i):
            cnt_v[pl.ds(i * 16, 16)] = zeros16

        start(0, 0)
        start(1, 1)

        def chunk_pair(ci, carry):
            for b in range(2):
                wait(b)

                @plsc.parallel_loop(0, CHUNK // 16, step=1, unroll=UNROLL)
                def _(i):
                    s = src_v[b, pl.ds(i * 16, 16)]
                    d = dst_v[b, pl.ds(i * 16, 16)]
                    r = d - row_base
                    m = (r >= 0) & (r < ROWS_PER_PASS)
                    fl = jnp.where(m, r * N + s, 0)
                    plsc.addupdate_scatter(cnt_v, [fl], ones, mask=m)

                @pl.when(ci + 2 + b < NCHUNK)
                def _():
                    start(ci + 2 + b, b)
            return carry

        lax.fori_loop(0, NCHUNK // 2, lambda i, c: chunk_pair(i * 2, c), 0)
        pltpu.sync_copy(
            cnt_v, c_hbm.at[pl.ds(row_base * N, ROWS_PER_PASS * N)])


def _build_counts(src, dst):
    mesh = plsc.VectorSubcoreMesh(
        core_axis_name="c", subcore_axis_name="s",
        num_cores=NC, num_subcores=NS)
    kern = pl.kernel(
        _count_body,
        out_type=jax.ShapeDtypeStruct((N * N,), jnp.float32),
        mesh=mesh,
        scratch_types=[
            pltpu.VMEM((2, CHUNK), jnp.int32),
            pltpu.VMEM((2, CHUNK), jnp.int32),
            pltpu.VMEM((ROWS_PER_PASS * N,), jnp.float32),
            pltpu.SemaphoreType.DMA((2,)),
            pltpu.SemaphoreType.DMA((2,)),
        ],
        compiler_params=pltpu.CompilerParams(needs_layout_passes=False),
    )
    return kern(src, dst).reshape(N, N)


def _prep1_body(x_ref, w_ref, amat_ref, h_ref, sc_ref):
    h = jnp.dot(x_ref[...], w_ref[...], preferred_element_type=jnp.float32)
    h_ref[...] = h
    sc_ref[...] = jnp.dot(h, amat_ref[...], preferred_element_type=jnp.float32,
                          precision=lax.Precision.HIGHEST)


def _prep_bn_body(raw_ref, g_ref, be_ref, w_ref, amat_ref, h_ref, sc_ref):
    r = raw_ref[...]
    m = jnp.mean(r, axis=0, keepdims=True)
    v = jnp.mean((r - m) * (r - m), axis=0, keepdims=True)
    xn = (r - m) * lax.rsqrt(v + 1e-5) * g_ref[...] + be_ref[...]
    xn = jnp.maximum(xn, 0.0)
    h = jnp.dot(xn, w_ref[...], preferred_element_type=jnp.float32)
    h_ref[...] = h
    sc_ref[...] = jnp.dot(h, amat_ref[...], preferred_element_type=jnp.float32,
                          precision=lax.Precision.HIGHEST)


def _prep(x, w, amat, g=None, be=None):
    n, f = x.shape
    body = _prep1_body if g is None else _prep_bn_body
    args = (x, w, amat) if g is None else (x, g.reshape(1, D), be.reshape(1, D), w, amat)
    return pl.pallas_call(
        body,
        out_shape=(jax.ShapeDtypeStruct((n, D), jnp.float32),
                   jax.ShapeDtypeStruct((n, 2 * NH), jnp.float32)),
    )(*args)


TD = 512


def _att_body(c_ref, sct_ref, sc_ref, h_ref, b_ref, out_ref):
    i = pl.program_id(0)
    cmat = c_ref[...]
    cols = []
    for hd in range(NH):
        arow = sct_ref[hd:hd + 1, :]
        acol = sc_ref[pl.ds(i * TD, TD), NH + hd:NH + hd + 1]
        mxs = jnp.max(arow, axis=1, keepdims=True)
        al = arow + acol
        al = jnp.where(al >= 0.0, al, 0.2 * al)
        amax = mxs + acol
        amax = jnp.where(amax >= 0.0, amax, 0.2 * amax)
        ex = cmat * jnp.exp(al - amax)
        denom = jnp.sum(ex, axis=1, keepdims=True)
        hf = h_ref[:, hd * HD:(hd + 1) * HD]
        hh = hf.astype(jnp.bfloat16)
        hl = (hf - hh.astype(jnp.float32)).astype(jnp.bfloat16)
        a16 = ex.astype(jnp.bfloat16)
        b16 = (ex - a16.astype(jnp.float32)).astype(jnp.bfloat16)
        num = (jnp.dot(a16, hh, preferred_element_type=jnp.float32)
               + jnp.dot(b16, hh, preferred_element_type=jnp.float32)
               + jnp.dot(a16, hl, preferred_element_type=jnp.float32))
        cols.append(num / (denom + 1e-16))
    out_ref[...] = jnp.concatenate(cols, axis=1) + b_ref[...]


def _attention(c, sct, sc, h, b):
    b = b.reshape(1, D)
    grid = (N // TD,)
    return pl.pallas_call(
        _att_body,
        grid=grid,
        in_specs=[
            pl.BlockSpec((TD, N), lambda i: (i, 0)),
            pl.BlockSpec((8, N), lambda i: (0, 0)),
            pl.BlockSpec((N, 2 * NH), lambda i: (0, 0)),
            pl.BlockSpec((N, D), lambda i: (0, 0)),
            pl.BlockSpec((1, D), lambda i: (0, 0)),
        ],
        out_specs=pl.BlockSpec((TD, D), lambda i: (i, 0)),
        out_shape=jax.ShapeDtypeStruct((N, D), jnp.float32),
    )(c, sct, sc, h, b)


def _final_body(raw_ref, g_ref, be_ref, wfc_ref, bfc_ref, mask_ref,
                tri_ref, adj_ref, sel_ref, selo_ref, out_ref):
    r = raw_ref[...]
    m = jnp.mean(r, axis=0, keepdims=True)
    v = jnp.mean((r - m) * (r - m), axis=0, keepdims=True)
    xn = (r - m) * lax.rsqrt(v + 1e-5) * g_ref[...] + be_ref[...]
    xn = jnp.maximum(xn, 0.0)
    t = jnp.dot(xn, wfc_ref[...], preferred_element_type=jnp.float32)
    t = (t + bfc_ref[...]) * mask_ref[...]
    tri = tri_ref[...]
    adj = adj_ref[...]
    a = [jnp.dot(t, sel_ref[k], preferred_element_type=jnp.float32) * tri
         for k in range(3)]
    mx = jnp.maximum(jnp.maximum(a[0], a[1]), a[2])
    e = [jnp.exp(ak - mx) for ak in a]
    s = e[0] + e[1] + e[2]
    p0 = jnp.where(adj == 0.0, 0.0, e[0] / s)
    p1 = e[1] / s
    p2 = jnp.where(adj == 1.0, 0.0, e[2] / s)
    tot = p0 + p1 + p2
    out = (jnp.dot(p0 / tot, selo_ref[0], preferred_element_type=jnp.float32)
           + jnp.dot(p1 / tot, selo_ref[1], preferred_element_type=jnp.float32)
           + jnp.dot(p2 / tot, selo_ref[2], preferred_element_type=jnp.float32))
    out_ref[...] = out


def _final(raw, g, be, wfc_p, bfc_p, mask2, tri_r, adj_r, sel, selo):
    return pl.pallas_call(
        _final_body,
        out_shape=jax.ShapeDtypeStruct((N, 128), jnp.float32),
    )(raw, g.reshape(1, D), be.reshape(1, D), wfc_p, bfc_p, mask2,
      tri_r, adj_r, sel, selo)


def _amat(a_src, a_dst):
    eye = jnp.eye(NH, dtype=jnp.float32)
    blk_s = jnp.einsum("hc,hk->hck", a_src.reshape(NH, HD), eye).reshape(D, NH)
    blk_d = jnp.einsum("hc,hk->hck", a_dst.reshape(NH, HD), eye).reshape(D, NH)
    return jnp.concatenate([blk_s, blk_d], axis=1)


def kernel(x, edge_index, pos, mask, adjacency, tri, W1, as1, ad1, b1, g1, be1,
           W2, as2, ad2, b2, g2, be2, W3, as3, ad3, b3, g3, be3, Wfc, bfc):
    src = edge_index[0]
    dst = edge_index[1]
    c = _build_counts(src, dst)

    sel_np = np.zeros((3, 128, 40), np.float32)
    for k in range(3):
        sel_np[k, np.arange(40) * 3 + k, np.arange(40)] = 1.0
    sel = jnp.asarray(sel_np)
    selo = jnp.asarray(np.transpose(sel_np, (0, 2, 1)))

    wfc_p = jnp.zeros((D, 128), jnp.float32).at[:, :120].set(Wfc)
    bfc_p = jnp.zeros((1, 128), jnp.float32).at[:, :120].set(bfc)
    mask2 = mask.reshape(N, 1)
    tri_r = tri.reshape(N, 40)
    adj_r = adjacency.astype(jnp.float32).reshape(N, 40)

    h, sc = _prep(x, W1, _amat(as1, ad1))
    sct = jnp.zeros((8, N), jnp.float32).at[:4, :].set(sc[:, :4].T)
    raw = _attention(c, sct, sc, h, b1)
    h, sc = _prep(raw, W2, _amat(as2, ad2), g1, be1)
    sct = jnp.zeros((8, N), jnp.float32).at[:4, :].set(sc[:, :4].T)
    raw = _attention(c, sct, sc, h, b2)
    h, sc = _prep(raw, W3, _amat(as3, ad3), g2, be2)
    sct = jnp.zeros((8, N), jnp.float32).at[:4, :].set(sc[:, :4].T)
    raw = _attention(c, sct, sc, h, b3)
    out = _final(raw, g3, be3, wfc_p, bfc_p, mask2, tri_r, adj_r, sel, selo)
    return out[:, :120].reshape(64, 40, 40, 3)
